# baseline (speedup 1.0000x reference)
import jax
import jax.numpy as jnp
from jax import lax
from jax.experimental import pallas as pl
from jax.experimental.pallas import tpu as pltpu

N_NODES = 10000
D_IN = 128
D_OUT = 128
BM = 200


def _gcn_block_kernel(a_ref, x_ref, w_ref, b_ref, o_ref, zbf_ref):
    @pl.when(pl.program_id(0) == 0)
    def _():
        z = lax.dot_general(
            x_ref[...], w_ref[...], (((1,), (1,)), ((), ())),
            preferred_element_type=jnp.float32,
        )
        zbf_ref[...] = z.astype(jnp.bfloat16)

    a_bf = a_ref[...].astype(jnp.bfloat16)
    out = jnp.dot(a_bf, zbf_ref[...], preferred_element_type=jnp.float32)
    o_ref[...] = out + b_ref[...]


def kernel(x, adj_normalized, W, b):
    b2 = b.reshape(1, D_OUT)
    grid = (N_NODES // BM,)
    out = pl.pallas_call(
        _gcn_block_kernel,
        grid=grid,
        in_specs=[
            pl.BlockSpec((BM, N_NODES), lambda i: (i, 0)),
            pl.BlockSpec((N_NODES, D_IN), lambda i: (0, 0)),
            pl.BlockSpec((D_OUT, D_IN), lambda i: (0, 0)),
            pl.BlockSpec((1, D_OUT), lambda i: (0, 0)),
        ],
        out_specs=pl.BlockSpec((BM, D_OUT), lambda i: (i, 0)),
        out_shape=jax.ShapeDtypeStruct((N_NODES, D_OUT), jnp.float32),
        scratch_shapes=[pltpu.VMEM((N_NODES, D_OUT), jnp.bfloat16)],
        compiler_params=pltpu.CompilerParams(vmem_limit_bytes=60 * 1024 * 1024),
    )(adj_normalized, x, W, b2)
    return out


# speedup vs baseline: 1.0126x; 1.0126x over previous
import jax
import jax.numpy as jnp
from jax import lax
from jax.experimental import pallas as pl
from jax.experimental.pallas import tpu as pltpu

N_NODES = 10000
D_IN = 128
D_OUT = 128
BM = 400


def _gcn_block_kernel(a_ref, x_ref, w_ref, b_ref, o_ref, zbf_ref):
    @pl.when(pl.program_id(0) == 0)
    def _():
        z = lax.dot_general(
            x_ref[...], w_ref[...], (((1,), (1,)), ((), ())),
            preferred_element_type=jnp.float32,
        )
        zbf_ref[...] = z

    out = jnp.dot(a_ref[...], zbf_ref[...], preferred_element_type=jnp.float32)
    o_ref[...] = out + b_ref[...]


def kernel(x, adj_normalized, W, b):
    b2 = b.reshape(1, D_OUT)
    grid = (N_NODES // BM,)
    out = pl.pallas_call(
        _gcn_block_kernel,
        grid=grid,
        in_specs=[
            pl.BlockSpec((BM, N_NODES), lambda i: (i, 0)),
            pl.BlockSpec((N_NODES, D_IN), lambda i: (0, 0)),
            pl.BlockSpec((D_OUT, D_IN), lambda i: (0, 0)),
            pl.BlockSpec((1, D_OUT), lambda i: (0, 0)),
        ],
        out_specs=pl.BlockSpec((BM, D_OUT), lambda i: (i, 0)),
        out_shape=jax.ShapeDtypeStruct((N_NODES, D_OUT), jnp.float32),
        scratch_shapes=[pltpu.VMEM((N_NODES, D_OUT), jnp.float32)],
        compiler_params=pltpu.CompilerParams(vmem_limit_bytes=60 * 1024 * 1024),
    )(adj_normalized, x, W, b2)
    return out


# speedup vs baseline: 1.0174x; 1.0048x over previous
import jax
import jax.numpy as jnp
from jax import lax
from jax.experimental import pallas as pl
from jax.experimental.pallas import tpu as pltpu

N_NODES = 10000
D_IN = 128
D_OUT = 128
BM = 400


def _gcn_block_kernel(a_ref, x_ref, w_ref, b_ref, o_ref, zbf_ref):
    @pl.when(pl.program_id(0) == 0)
    def _():
        z = lax.dot_general(
            x_ref[...], w_ref[...], (((1,), (1,)), ((), ())),
            preferred_element_type=jnp.float32,
        )
        zbf_ref[...] = z.astype(jnp.bfloat16)

    a_bf = a_ref[...].astype(jnp.bfloat16)
    out = jnp.dot(a_bf, zbf_ref[...], preferred_element_type=jnp.float32)
    o_ref[...] = out + b_ref[...]


def kernel(x, adj_normalized, W, b):
    b2 = b.reshape(1, D_OUT)
    grid = (N_NODES // BM,)
    out = pl.pallas_call(
        _gcn_block_kernel,
        grid=grid,
        in_specs=[
            pl.BlockSpec((BM, N_NODES), lambda i: (i, 0)),
            pl.BlockSpec((N_NODES, D_IN), lambda i: (0, 0)),
            pl.BlockSpec((D_OUT, D_IN), lambda i: (0, 0)),
            pl.BlockSpec((1, D_OUT), lambda i: (0, 0)),
        ],
        out_specs=pl.BlockSpec((BM, D_OUT), lambda i: (i, 0)),
        out_shape=jax.ShapeDtypeStruct((N_NODES, D_OUT), jnp.float32),
        scratch_shapes=[pltpu.VMEM((N_NODES, D_OUT), jnp.bfloat16)],
        compiler_params=pltpu.CompilerParams(vmem_limit_bytes=60 * 1024 * 1024),
    )(adj_normalized, x, W, b2)
    return out
